# Initial kernel scaffold; baseline (speedup 1.0000x reference)
import jax
import jax.numpy as jnp
from jax.experimental import pallas as pl


def kernel(node_feat, edge_index, edge_feat, W_A, b_A, W_B, b_B, W_ih1, W_hh1, b_ih1, b_hh1, W_ih2, W_hh2, b_ih2, b_hh2, gamma_h, beta_h, gamma_e, beta_e):
    raise NotImplementedError("write your pallas kernel here")


# speedup vs baseline: 1.1114x; 1.1114x over previous
import functools

import jax
import jax.numpy as jnp
from jax.experimental import pallas as pl
from jax.experimental.pallas import tpu as pltpu

_D = 96


def _ab_body(x_ref, wa_ref, ba_ref, wb_ref, bb_ref, ah_ref, bh_ref):
    x = x_ref[...]
    ah_ref[...] = jnp.dot(x, wa_ref[...], preferred_element_type=jnp.float32) + ba_ref[...]
    bh_ref[...] = jnp.dot(x, wb_ref[...], preferred_element_type=jnp.float32) + bb_ref[...]


def _node_body(nf_ref, ah_ref, num_ref, den_ref, h_ref, stats_ref):
    i = pl.program_id(0)
    den = den_ref[...]
    h_agg = ah_ref[...] + num_ref[...] / (den + 1e-6)
    mask = den[:, 0:1] > 0.0
    h = jnp.where(mask, h_agg, nf_ref[...])
    h_ref[...] = h

    @pl.when(i == 0)
    def _():
        stats_ref[...] = jnp.zeros_like(stats_ref)

    stats_ref[0:1, :] += jnp.sum(h, axis=0, keepdims=True)
    stats_ref[1:2, :] += jnp.sum(h * h, axis=0, keepdims=True)


def _gru_body(e_ref, sh_ref, dh_ref, w_ref, b_ref, e2_ref, stats_ref):
    i = pl.program_id(0)
    e = e_ref[...]
    sh = sh_ref[...]
    dh = dh_ref[...]

    def mm(x, k):
        return jnp.dot(x, w_ref[k], preferred_element_type=jnp.float32)

    def bias(k):
        return b_ref[k : k + 1, :]

    r1 = jax.nn.sigmoid(mm(sh, 0) + bias(0) + mm(e, 3) + bias(3))
    z1 = jax.nn.sigmoid(mm(sh, 1) + bias(1) + mm(e, 4) + bias(4))
    n1 = jnp.tanh(mm(sh, 2) + bias(2) + r1 * (mm(e, 5) + bias(5)))
    e1 = (1.0 - z1) * n1 + z1 * e

    r2 = jax.nn.sigmoid(mm(dh, 6) + bias(6) + mm(e1, 9) + bias(9))
    z2 = jax.nn.sigmoid(mm(dh, 7) + bias(7) + mm(e1, 10) + bias(10))
    n2 = jnp.tanh(mm(dh, 8) + bias(8) + r2 * (mm(e1, 11) + bias(11)))
    e2 = (1.0 - z2) * n2 + z2 * e1

    e2_ref[...] = e2

    @pl.when(i == 0)
    def _():
        stats_ref[...] = jnp.zeros_like(stats_ref)

    stats_ref[0:1, :] += jnp.sum(e2, axis=0, keepdims=True)
    stats_ref[1:2, :] += jnp.sum(e2 * e2, axis=0, keepdims=True)


def _apply_body(x_ref, res_ref, scale_ref, shift_ref, o_ref):
    o_ref[...] = res_ref[...] + jnp.maximum(
        x_ref[...] * scale_ref[...] + shift_ref[...], 0.0
    )


def _full_spec():
    return pl.BlockSpec(lambda i: (0, 0))


def _row_spec(blk):
    return pl.BlockSpec((blk, _D), lambda i: (i, 0))


def _compute_ab(node_feat, W_A, b_A, W_B, b_B, blk):
    n = node_feat.shape[0]
    grid = n // blk
    return pl.pallas_call(
        _ab_body,
        grid=(grid,),
        in_specs=[
            _row_spec(blk),
            pl.BlockSpec((_D, _D), lambda i: (0, 0)),
            pl.BlockSpec((1, _D), lambda i: (0, 0)),
            pl.BlockSpec((_D, _D), lambda i: (0, 0)),
            pl.BlockSpec((1, _D), lambda i: (0, 0)),
        ],
        out_specs=[_row_spec(blk), _row_spec(blk)],
        out_shape=[
            jax.ShapeDtypeStruct((n, _D), jnp.float32),
            jax.ShapeDtypeStruct((n, _D), jnp.float32),
        ],
    )(node_feat, W_A.T, b_A.reshape(1, _D), W_B.T, b_B.reshape(1, _D))


def _node_update(node_feat, ah, num, den, blk):
    n = node_feat.shape[0]
    grid = n // blk
    return pl.pallas_call(
        _node_body,
        grid=(grid,),
        in_specs=[_row_spec(blk)] * 4,
        out_specs=[_row_spec(blk), pl.BlockSpec((8, _D), lambda i: (0, 0))],
        out_shape=[
            jax.ShapeDtypeStruct((n, _D), jnp.float32),
            jax.ShapeDtypeStruct((8, _D), jnp.float32),
        ],
    )(node_feat, ah, num, den)


def _gru_chain(e, sh, dh, ws, bs, blk):
    m = e.shape[0]
    grid = m // blk
    return pl.pallas_call(
        _gru_body,
        grid=(grid,),
        in_specs=[
            _row_spec(blk),
            _row_spec(blk),
            _row_spec(blk),
            pl.BlockSpec((12, _D, _D), lambda i: (0, 0, 0)),
            pl.BlockSpec((16, _D), lambda i: (0, 0)),
        ],
        out_specs=[_row_spec(blk), pl.BlockSpec((8, _D), lambda i: (0, 0))],
        out_shape=[
            jax.ShapeDtypeStruct((m, _D), jnp.float32),
            jax.ShapeDtypeStruct((8, _D), jnp.float32),
        ],
    )(e, sh, dh, ws, bs)


def _bn_apply(x, res, scale, shift, blk):
    m = x.shape[0]
    grid = m // blk
    return pl.pallas_call(
        _apply_body,
        grid=(grid,),
        in_specs=[
            _row_spec(blk),
            _row_spec(blk),
            pl.BlockSpec((1, _D), lambda i: (0, 0)),
            pl.BlockSpec((1, _D), lambda i: (0, 0)),
        ],
        out_specs=_row_spec(blk),
        out_shape=jax.ShapeDtypeStruct((m, _D), jnp.float32),
    )(x, res, scale, shift)


def _bn_coeffs(stats, count, gamma, beta):
    mean = stats[0, :] / count
    var = stats[1, :] / count - mean * mean
    inv = jax.lax.rsqrt(var + 1e-5)
    scale = gamma * inv
    shift = beta - mean * scale
    return scale.reshape(1, _D), shift.reshape(1, _D)


def kernel(node_feat, edge_index, edge_feat, W_A, b_A, W_B, b_B,
           W_ih1, W_hh1, b_ih1, b_hh1, W_ih2, W_hh2, b_ih2, b_hh2,
           gamma_h, beta_h, gamma_e, beta_e):
    n = node_feat.shape[0]
    m = edge_feat.shape[0]
    nblk = 5000 if n % 5000 == 0 else 8
    eblk = 4000 if m % 4000 == 0 else 8

    src = edge_index[0]
    dst = edge_index[1]

    ah, bh = _compute_ab(node_feat, W_A, b_A, W_B, b_B, nblk)

    sigma = jax.nn.sigmoid(edge_feat)
    bh_j = jnp.take(bh, src, axis=0)
    num = jax.ops.segment_sum(sigma * bh_j, dst, num_segments=n)
    den = jax.ops.segment_sum(sigma, dst, num_segments=n)

    h_pre, h_stats = _node_update(node_feat, ah, num, den, nblk)

    sh = jnp.take(h_pre, src, axis=0)
    dh = jnp.take(h_pre, dst, axis=0)

    def split3(w):
        return [w[0:_D].T, w[_D:2 * _D].T, w[2 * _D:3 * _D].T]

    ws = jnp.stack(split3(W_ih1) + split3(W_hh1) + split3(W_ih2) + split3(W_hh2))
    bs = jnp.concatenate([
        b_ih1.reshape(3, _D), b_hh1.reshape(3, _D),
        b_ih2.reshape(3, _D), b_hh2.reshape(3, _D),
        jnp.zeros((4, _D), jnp.float32),
    ])

    e2, e_stats = _gru_chain(edge_feat, sh, dh, ws, bs, eblk)

    h_scale, h_shift = _bn_coeffs(h_stats, jnp.float32(n), gamma_h, beta_h)
    e_scale, e_shift = _bn_coeffs(e_stats, jnp.float32(m), gamma_e, beta_e)

    h_out = _bn_apply(h_pre, node_feat, h_scale, h_shift, nblk)
    e_out = _bn_apply(e2, edge_feat, e_scale, e_shift, eblk)
    return (h_out, e_out)


# speedup vs baseline: 1.6864x; 1.5174x over previous
import functools

import jax
import jax.numpy as jnp
from jax import lax
from jax.experimental import pallas as pl
from jax.experimental.pallas import tpu as pltpu
from jax.experimental.pallas import tpu_sc as plsc

_D = 96
_DP = 128
_NC = 2
_NS = 16
_NW = _NC * _NS


def _sc_gather2(table, idx_a, idx_b, chunk):
    e = idx_a.shape[0]
    d = table.shape[1]
    bpw = e // _NW
    n_iter = bpw // chunk
    mesh = plsc.VectorSubcoreMesh(core_axis_name="c", subcore_axis_name="s")

    @functools.partial(
        pl.kernel,
        mesh=mesh,
        out_type=[
            jax.ShapeDtypeStruct((e, d), jnp.float32),
            jax.ShapeDtypeStruct((e, d), jnp.float32),
        ],
        scratch_types=[
            pltpu.VMEM((chunk,), jnp.int32),
            pltpu.VMEM((chunk,), jnp.int32),
            pltpu.VMEM((chunk, d), jnp.float32),
            pltpu.VMEM((chunk, d), jnp.float32),
            pltpu.SemaphoreType.DMA,
            pltpu.SemaphoreType.DMA,
        ],
    )
    def k(table_hbm, ia_hbm, ib_hbm, oa_hbm, ob_hbm,
          ia_v, ib_v, ra_v, rb_v, sem_a, sem_b):
        wid = lax.axis_index("s") * _NC + lax.axis_index("c")
        base = wid * bpw

        def body(i, _):
            off = base + i * chunk
            pltpu.sync_copy(ia_hbm.at[pl.ds(off, chunk)], ia_v)
            pltpu.sync_copy(ib_hbm.at[pl.ds(off, chunk)], ib_v)
            ca = pltpu.async_copy(table_hbm.at[ia_v], ra_v, sem_a)
            cb = pltpu.async_copy(table_hbm.at[ib_v], rb_v, sem_b)
            ca.wait()
            pltpu.sync_copy(ra_v, oa_hbm.at[pl.ds(off, chunk)])
            cb.wait()
            pltpu.sync_copy(rb_v, ob_hbm.at[pl.ds(off, chunk)])
            return 0

        lax.fori_loop(0, n_iter, body, 0)

    return k(table, idx_a, idx_b)


def _sc_gather1(table, idx, chunk):
    e = idx.shape[0]
    d = table.shape[1]
    bpw = e // _NW
    n_iter = bpw // chunk
    mesh = plsc.VectorSubcoreMesh(core_axis_name="c", subcore_axis_name="s")

    @functools.partial(
        pl.kernel,
        mesh=mesh,
        out_type=jax.ShapeDtypeStruct((e, d), jnp.float32),
        scratch_types=[
            pltpu.VMEM((chunk,), jnp.int32),
            pltpu.VMEM((chunk, d), jnp.float32),
            pltpu.SemaphoreType.DMA,
        ],
    )
    def k(table_hbm, i_hbm, o_hbm, i_v, r_v, sem):
        wid = lax.axis_index("s") * _NC + lax.axis_index("c")
        base = wid * bpw

        def body(i, _):
            off = base + i * chunk
            pltpu.sync_copy(i_hbm.at[pl.ds(off, chunk)], i_v)
            pltpu.async_copy(table_hbm.at[i_v], r_v, sem).wait()
            pltpu.sync_copy(r_v, o_hbm.at[pl.ds(off, chunk)])
            return 0

        lax.fori_loop(0, n_iter, body, 0)

    return k(table, idx)


def _ab_body(x_ref, wa_ref, ba_ref, wb_ref, bb_ref, ah_ref, bh_ref):
    x = x_ref[...]
    ah_ref[...] = jnp.dot(x, wa_ref[...], preferred_element_type=jnp.float32) + ba_ref[...]
    bh_ref[...] = jnp.dot(x, wb_ref[...], preferred_element_type=jnp.float32) + bb_ref[...]


def _node_body(nf_ref, ah_ref, num_ref, den_ref, h_ref, stats_ref):
    i = pl.program_id(0)
    den = den_ref[...]
    pad = jnp.zeros((nf_ref.shape[0], _DP - _D), jnp.float32)
    h_agg = ah_ref[...] + jnp.concatenate(
        [num_ref[...] / (den + 1e-6), pad], axis=1)
    mask = den[:, 0:1] > 0.0
    h = jnp.where(mask, h_agg, jnp.concatenate([nf_ref[...], pad], axis=1))
    h_ref[...] = h

    @pl.when(i == 0)
    def _():
        stats_ref[...] = jnp.zeros_like(stats_ref)

    stats_ref[0:1, :] += jnp.sum(h, axis=0, keepdims=True)
    stats_ref[1:2, :] += jnp.sum(h * h, axis=0, keepdims=True)


def _gru_body(e_ref, sh_ref, dh_ref, w_ref, b_ref, e2_ref, stats_ref):
    i = pl.program_id(0)
    e = e_ref[...]
    sh = sh_ref[:, :_D]
    dh = dh_ref[:, :_D]

    def mm(x, k):
        return jnp.dot(x, w_ref[k], preferred_element_type=jnp.float32)

    def bias(k):
        return b_ref[k : k + 1, :]

    r1 = jax.nn.sigmoid(mm(sh, 0) + bias(0) + mm(e, 3) + bias(3))
    z1 = jax.nn.sigmoid(mm(sh, 1) + bias(1) + mm(e, 4) + bias(4))
    n1 = jnp.tanh(mm(sh, 2) + bias(2) + r1 * (mm(e, 5) + bias(5)))
    e1 = (1.0 - z1) * n1 + z1 * e

    r2 = jax.nn.sigmoid(mm(dh, 6) + bias(6) + mm(e1, 9) + bias(9))
    z2 = jax.nn.sigmoid(mm(dh, 7) + bias(7) + mm(e1, 10) + bias(10))
    n2 = jnp.tanh(mm(dh, 8) + bias(8) + r2 * (mm(e1, 11) + bias(11)))
    e2 = (1.0 - z2) * n2 + z2 * e1

    e2_ref[...] = e2

    @pl.when(i == 0)
    def _():
        stats_ref[...] = jnp.zeros_like(stats_ref)

    stats_ref[0:1, :] += jnp.sum(e2, axis=0, keepdims=True)
    stats_ref[1:2, :] += jnp.sum(e2 * e2, axis=0, keepdims=True)


def _apply_body(x_ref, res_ref, scale_ref, shift_ref, o_ref):
    x = x_ref[:, :_D]
    o_ref[...] = res_ref[...] + jnp.maximum(
        x * scale_ref[...] + shift_ref[...], 0.0
    )


def _full_spec():
    return pl.BlockSpec(lambda i: (0, 0))


def _row_spec(blk):
    return pl.BlockSpec((blk, _D), lambda i: (i, 0))


def _compute_ab(node_feat, W_A, b_A, W_B, b_B, blk):
    n = node_feat.shape[0]
    grid = n // blk
    padw = jnp.zeros((_D, _DP - _D), jnp.float32)
    padb = jnp.zeros((1, _DP - _D), jnp.float32)
    wa = jnp.concatenate([W_A.T, padw], axis=1)
    wb = jnp.concatenate([W_B.T, padw], axis=1)
    ba = jnp.concatenate([b_A.reshape(1, _D), padb], axis=1)
    bb = jnp.concatenate([b_B.reshape(1, _D), padb], axis=1)
    return pl.pallas_call(
        _ab_body,
        grid=(grid,),
        in_specs=[
            _row_spec(blk),
            pl.BlockSpec((_D, _DP), lambda i: (0, 0)),
            pl.BlockSpec((1, _DP), lambda i: (0, 0)),
            pl.BlockSpec((_D, _DP), lambda i: (0, 0)),
            pl.BlockSpec((1, _DP), lambda i: (0, 0)),
        ],
        out_specs=[
            pl.BlockSpec((blk, _DP), lambda i: (i, 0)),
            pl.BlockSpec((blk, _DP), lambda i: (i, 0)),
        ],
        out_shape=[
            jax.ShapeDtypeStruct((n, _DP), jnp.float32),
            jax.ShapeDtypeStruct((n, _DP), jnp.float32),
        ],
    )(node_feat, wa, ba, wb, bb)


def _node_update(node_feat, ah, num, den, blk):
    n = node_feat.shape[0]
    grid = n // blk
    return pl.pallas_call(
        _node_body,
        grid=(grid,),
        in_specs=[
            _row_spec(blk),
            pl.BlockSpec((blk, _DP), lambda i: (i, 0)),
            _row_spec(blk),
            _row_spec(blk),
        ],
        out_specs=[
            pl.BlockSpec((blk, _DP), lambda i: (i, 0)),
            pl.BlockSpec((8, _DP), lambda i: (0, 0)),
        ],
        out_shape=[
            jax.ShapeDtypeStruct((n, _DP), jnp.float32),
            jax.ShapeDtypeStruct((8, _DP), jnp.float32),
        ],
    )(node_feat, ah, num, den)


def _gru_chain(e, sh, dh, ws, bs, blk):
    m = e.shape[0]
    grid = m // blk
    return pl.pallas_call(
        _gru_body,
        grid=(grid,),
        in_specs=[
            _row_spec(blk),
            pl.BlockSpec((blk, _DP), lambda i: (i, 0)),
            pl.BlockSpec((blk, _DP), lambda i: (i, 0)),
            pl.BlockSpec((12, _D, _D), lambda i: (0, 0, 0)),
            pl.BlockSpec((16, _D), lambda i: (0, 0)),
        ],
        out_specs=[_row_spec(blk), pl.BlockSpec((8, _D), lambda i: (0, 0))],
        out_shape=[
            jax.ShapeDtypeStruct((m, _D), jnp.float32),
            jax.ShapeDtypeStruct((8, _D), jnp.float32),
        ],
    )(e, sh, dh, ws, bs)


def _bn_apply(x, res, scale, shift, blk):
    m, xw = x.shape
    grid = m // blk
    return pl.pallas_call(
        _apply_body,
        grid=(grid,),
        in_specs=[
            pl.BlockSpec((blk, xw), lambda i: (i, 0)),
            _row_spec(blk),
            pl.BlockSpec((1, _D), lambda i: (0, 0)),
            pl.BlockSpec((1, _D), lambda i: (0, 0)),
        ],
        out_specs=_row_spec(blk),
        out_shape=jax.ShapeDtypeStruct((m, _D), jnp.float32),
    )(x, res, scale, shift)


def _bn_coeffs(stats, count, gamma, beta):
    mean = stats[0, :] / count
    var = stats[1, :] / count - mean * mean
    inv = jax.lax.rsqrt(var + 1e-5)
    scale = gamma * inv
    shift = beta - mean * scale
    return scale.reshape(1, _D), shift.reshape(1, _D)


def kernel(node_feat, edge_index, edge_feat, W_A, b_A, W_B, b_B,
           W_ih1, W_hh1, b_ih1, b_hh1, W_ih2, W_hh2, b_ih2, b_hh2,
           gamma_h, beta_h, gamma_e, beta_e):
    n = node_feat.shape[0]
    m = edge_feat.shape[0]
    nblk = 5000 if n % 5000 == 0 else 8
    eblk = 4000 if m % 4000 == 0 else 8

    src = edge_index[0]
    dst = edge_index[1]

    ah, bh = _compute_ab(node_feat, W_A, b_A, W_B, b_B, nblk)

    sigma = jax.nn.sigmoid(edge_feat)
    bh_j = _sc_gather1(bh, src, 200)
    num = jax.ops.segment_sum(sigma * bh_j[:, :_D], dst, num_segments=n)
    den = jax.ops.segment_sum(sigma, dst, num_segments=n)

    h_pre, h_stats = _node_update(node_feat, ah, num, den, nblk)

    sh, dh = _sc_gather2(h_pre, src, dst, 200)

    def split3(w):
        return [w[0:_D].T, w[_D:2 * _D].T, w[2 * _D:3 * _D].T]

    ws = jnp.stack(split3(W_ih1) + split3(W_hh1) + split3(W_ih2) + split3(W_hh2))
    bs = jnp.concatenate([
        b_ih1.reshape(3, _D), b_hh1.reshape(3, _D),
        b_ih2.reshape(3, _D), b_hh2.reshape(3, _D),
        jnp.zeros((4, _D), jnp.float32),
    ])

    e2, e_stats = _gru_chain(edge_feat, sh, dh, ws, bs, eblk)

    h_scale, h_shift = _bn_coeffs(h_stats[:, :_D], jnp.float32(n), gamma_h, beta_h)
    e_scale, e_shift = _bn_coeffs(e_stats, jnp.float32(m), gamma_e, beta_e)

    h_out = _bn_apply(h_pre, node_feat, h_scale, h_shift, nblk)
    e_out = _bn_apply(e2, edge_feat, e_scale, e_shift, eblk)
    return (h_out, e_out)
